# baseline (speedup 1.0000x reference)
import functools

import jax
import jax.numpy as jnp
from jax import lax
from jax.experimental import pallas as pl
from jax.experimental.pallas import tpu as pltpu
from jax.experimental.pallas import tpu_sc as plsc

N = 10000
E = 320000
D = 128
NC = 2
NS = 16
NW = NC * NS
G = 80
RG = 2 * E // G
RPT = RG // NW
NCH = 10
CPR = RPT // NCH
G2 = 128
NCH2 = 4
CPR2 = 40
EP = NW * NCH2 * CPR2 * G2
NDUMMY = 240
NPAD = 10240
DPT = NPAD // NS
ZROWS = NPAD // NS
ZB = 32

_mesh = plsc.VectorSubcoreMesh(
    core_axis_name="c", subcore_axis_name="s", num_cores=NC, num_subcores=NS)


@functools.partial(
    pl.kernel,
    mesh=_mesh,
    out_type=jax.ShapeDtypeStruct((NC, 1, NPAD), jnp.float32),
    scratch_types=[
        pltpu.VMEM((RPT, G), jnp.int32),
        pltpu.VMEM((G,), jnp.float32),
        pltpu.VMEM((DPT,), jnp.float32),
        pltpu.VMEM_SHARED((NPAD,), jnp.float32),
    ],
)
def _deg_call(ridx_hbm, deg_hbm, idx_v, ones_v, zero_v, deg_sh):
    cid = lax.axis_index("c")
    sid = lax.axis_index("s")
    wid = cid * NS + sid

    for k in range(DPT // 16):
        zero_v[pl.ds(k * 16, 16)] = jnp.zeros((16,), jnp.float32)
    for k in range(G // 16):
        ones_v[pl.ds(k * 16, 16)] = jnp.ones((16,), jnp.float32)
    pltpu.sync_copy(zero_v, deg_sh.at[pl.ds(sid * DPT, DPT)])
    plsc.subcore_barrier()

    pltpu.sync_copy(ridx_hbm.at[wid], idx_v)

    @pl.loop(0, RPT)
    def _(g):
        pltpu.sync_copy(ones_v, deg_sh.at[idx_v.at[g]], add=True)

    plsc.subcore_barrier()
    pltpu.sync_copy(deg_sh.at[pl.ds(sid * DPT, DPT)],
                    deg_hbm.at[cid, 0, pl.ds(sid * DPT, DPT)])


@functools.partial(
    pl.kernel,
    mesh=_mesh,
    out_type=jax.ShapeDtypeStruct((NC, NPAD, D), jnp.float32),
    scratch_types=[
        pltpu.VMEM((CPR2, G2), jnp.int32),
        pltpu.VMEM((CPR2, G2), jnp.int32),
        pltpu.VMEM((G2, D), jnp.float32),
        pltpu.VMEM((G2, D), jnp.float32),
        pltpu.VMEM((ZB, D), jnp.float32),
        pltpu.VMEM_SHARED((NPAD, D), jnp.float32),
        pltpu.SemaphoreType.DMA,
        pltpu.SemaphoreType.DMA,
    ],
)
def _agg_call(z_hbm, gidx_hbm, sidx_hbm, part_hbm,
              gidx_v, sidx_v, rows0, rows1, zbuf_v, acc_sh, sem0, sem1):
    cid = lax.axis_index("c")
    sid = lax.axis_index("s")
    wid = cid * NS + sid

    @pl.loop(0, ZB)
    def _(i):
        for k in range(D // 16):
            zbuf_v[i, pl.ds(k * 16, 16)] = jnp.zeros((16,), jnp.float32)

    for k in range(ZROWS // ZB):
        pltpu.sync_copy(zbuf_v, acc_sh.at[pl.ds(sid * ZROWS + k * ZB, ZB)])
    plsc.subcore_barrier()

    def wait0():
        pltpu.make_async_copy(z_hbm.at[pl.ds(0, G2)], rows0, sem0).wait()

    def wait1():
        pltpu.make_async_copy(z_hbm.at[pl.ds(0, G2)], rows1, sem1).wait()

    @pl.loop(0, NCH2)
    def _(c):
        pltpu.sync_copy(gidx_hbm.at[wid, c], gidx_v)
        pltpu.sync_copy(sidx_hbm.at[wid, c], sidx_v)
        pltpu.async_copy(z_hbm.at[gidx_v.at[0]], rows0, sem0)

        @pl.loop(0, CPR2 // 2 - 1)
        def _(t):
            s0 = 2 * t
            pltpu.async_copy(z_hbm.at[gidx_v.at[s0 + 1]], rows1, sem1)
            wait0()
            pltpu.sync_copy(rows0, acc_sh.at[sidx_v.at[s0]], add=True)
            pltpu.async_copy(z_hbm.at[gidx_v.at[s0 + 2]], rows0, sem0)
            wait1()
            pltpu.sync_copy(rows1, acc_sh.at[sidx_v.at[s0 + 1]], add=True)

        pltpu.async_copy(z_hbm.at[gidx_v.at[CPR2 - 1]], rows1, sem1)
        wait0()
        pltpu.sync_copy(rows0, acc_sh.at[sidx_v.at[CPR2 - 2]], add=True)
        wait1()
        pltpu.sync_copy(rows1, acc_sh.at[sidx_v.at[CPR2 - 1]], add=True)

    plsc.subcore_barrier()
    pltpu.sync_copy(acc_sh.at[pl.ds(sid * ZROWS, ZROWS)],
                    part_hbm.at[cid, pl.ds(sid * ZROWS, ZROWS)])


BM = 1000


def _mm_body(x_ref, w_ref, deg_ref, z_ref):
    total = deg_ref[:, 0:1] + deg_ref[:, 1:2] + 1.0
    dis = lax.rsqrt(total)
    h = lax.dot_general(x_ref[...], w_ref[...], (((1,), (1,)), ((), ())),
                        preferred_element_type=jnp.float32)
    z_ref[...] = h * dis


_mm_call = pl.pallas_call(
    _mm_body,
    grid=(N // BM,),
    in_specs=[
        pl.BlockSpec((BM, D), lambda i: (i, 0)),
        pl.BlockSpec((D, D), lambda i: (0, 0)),
        pl.BlockSpec((BM, NC), lambda i: (i, 0)),
    ],
    out_specs=pl.BlockSpec((BM, D), lambda i: (i, 0)),
    out_shape=jax.ShapeDtypeStruct((N, D), jnp.float32),
)


def _fin_body(p0_ref, p1_ref, z_ref, deg_ref, b_ref, y_ref):
    total = deg_ref[:, 0:1] + deg_ref[:, 1:2] + 1.0
    dis = lax.rsqrt(total)
    y_ref[...] = dis * (p0_ref[...] + p1_ref[...] + z_ref[...]) + b_ref[...]


_fin_call = pl.pallas_call(
    _fin_body,
    grid=(N // BM,),
    in_specs=[
        pl.BlockSpec((BM, D), lambda i: (i, 0)),
        pl.BlockSpec((BM, D), lambda i: (i, 0)),
        pl.BlockSpec((BM, D), lambda i: (i, 0)),
        pl.BlockSpec((BM, NC), lambda i: (i, 0)),
        pl.BlockSpec((1, D), lambda i: (0, 0)),
    ],
    out_specs=pl.BlockSpec((BM, D), lambda i: (i, 0)),
    out_shape=jax.ShapeDtypeStruct((N, D), jnp.float32),
)


def kernel(x, edge_index, W, b):
    src = edge_index[0]
    dst = edge_index[1]
    sidx3d = edge_index.reshape(NW, RPT, G)
    npad_e = EP - 2 * E
    gpad = (jnp.arange(npad_e, dtype=jnp.int32) * 64) % N
    spad = N + jnp.arange(npad_e, dtype=jnp.int32) % NDUMMY
    gidx4d = jnp.concatenate([dst, src, gpad]).reshape(NW, NCH2, CPR2, G2)
    sidx4d = jnp.concatenate([src, dst, spad]).reshape(NW, NCH2, CPR2, G2)

    deg_pad = _deg_call(sidx3d)
    deg_pair = jnp.transpose(deg_pad[:, 0, :N])
    z = _mm_call(x, W, deg_pair)
    part = _agg_call(z, gidx4d, sidx4d)
    y = _fin_call(part[0, :N], part[1, :N], z, deg_pair, b.reshape(1, D))
    return y


# speedup vs baseline: 1.0392x; 1.0392x over previous
import functools

import jax
import jax.numpy as jnp
from jax import lax
from jax.experimental import pallas as pl
from jax.experimental.pallas import tpu as pltpu
from jax.experimental.pallas import tpu_sc as plsc

N = 10000
E = 320000
D = 128
NC = 2
NS = 16
NW = NC * NS
G = 80
RG = 2 * E // G
RPT = RG // NW
NCH = 10
CPR = RPT // NCH
G2 = 128
NCH2 = 4
CPR2 = 40
EP = NW * NCH2 * CPR2 * G2
NDUMMY = 240
NPAD = 10240
DPT = NPAD // NS
ZROWS = NPAD // NS
ZB = 32

_mesh = plsc.VectorSubcoreMesh(
    core_axis_name="c", subcore_axis_name="s", num_cores=NC, num_subcores=NS)


@functools.partial(
    pl.kernel,
    mesh=_mesh,
    out_type=jax.ShapeDtypeStruct((NC, 1, NPAD), jnp.float32),
    scratch_types=[
        pltpu.VMEM((CPR2, G2), jnp.int32),
        pltpu.VMEM((G2,), jnp.float32),
        pltpu.VMEM((DPT,), jnp.float32),
        pltpu.VMEM_SHARED((NPAD,), jnp.float32),
        pltpu.SemaphoreType.DMA,
    ],
)
def _deg_call(sidx_hbm, deg_hbm, idx_v, ones_v, zero_v, deg_sh, sem):
    cid = lax.axis_index("c")
    sid = lax.axis_index("s")
    wid = cid * NS + sid

    for k in range(DPT // 16):
        zero_v[pl.ds(k * 16, 16)] = jnp.zeros((16,), jnp.float32)
    for k in range(G2 // 16):
        ones_v[pl.ds(k * 16, 16)] = jnp.ones((16,), jnp.float32)
    pltpu.sync_copy(zero_v, deg_sh.at[pl.ds(sid * DPT, DPT)])
    plsc.subcore_barrier()

    @pl.loop(0, NCH2)
    def _(c):
        pltpu.sync_copy(sidx_hbm.at[wid, c], idx_v)

        @pl.loop(0, CPR2)
        def _(g):
            pltpu.async_copy(ones_v, deg_sh.at[idx_v.at[g]], sem, add=True)

        @pl.loop(0, CPR2)
        def _(g):
            pltpu.make_async_copy(ones_v, deg_sh.at[pl.ds(0, G2)], sem).wait()

    plsc.subcore_barrier()
    pltpu.sync_copy(deg_sh.at[pl.ds(sid * DPT, DPT)],
                    deg_hbm.at[cid, 0, pl.ds(sid * DPT, DPT)])


@functools.partial(
    pl.kernel,
    mesh=_mesh,
    out_type=jax.ShapeDtypeStruct((NC, NPAD, D), jnp.float32),
    scratch_types=[
        pltpu.VMEM((CPR2, G2), jnp.int32),
        pltpu.VMEM((CPR2, G2), jnp.int32),
        pltpu.VMEM((G2, D), jnp.float32),
        pltpu.VMEM((G2, D), jnp.float32),
        pltpu.VMEM((ZB, D), jnp.float32),
        pltpu.VMEM_SHARED((NPAD, D), jnp.float32),
        pltpu.SemaphoreType.DMA,
        pltpu.SemaphoreType.DMA,
    ],
)
def _agg_call(z_hbm, gidx_hbm, sidx_hbm, part_hbm,
              gidx_v, sidx_v, rows0, rows1, zbuf_v, acc_sh, sem0, sem1):
    cid = lax.axis_index("c")
    sid = lax.axis_index("s")
    wid = cid * NS + sid

    @pl.loop(0, ZB)
    def _(i):
        for k in range(D // 16):
            zbuf_v[i, pl.ds(k * 16, 16)] = jnp.zeros((16,), jnp.float32)

    for k in range(ZROWS // ZB):
        pltpu.sync_copy(zbuf_v, acc_sh.at[pl.ds(sid * ZROWS + k * ZB, ZB)])
    plsc.subcore_barrier()

    def wait0():
        pltpu.make_async_copy(z_hbm.at[pl.ds(0, G2)], rows0, sem0).wait()

    def wait1():
        pltpu.make_async_copy(z_hbm.at[pl.ds(0, G2)], rows1, sem1).wait()

    @pl.loop(0, NCH2)
    def _(c):
        pltpu.sync_copy(gidx_hbm.at[wid, c], gidx_v)
        pltpu.sync_copy(sidx_hbm.at[wid, c], sidx_v)
        pltpu.async_copy(z_hbm.at[gidx_v.at[0]], rows0, sem0)

        @pl.loop(0, CPR2 // 2 - 1)
        def _(t):
            s0 = 2 * t
            pltpu.async_copy(z_hbm.at[gidx_v.at[s0 + 1]], rows1, sem1)
            wait0()
            pltpu.sync_copy(rows0, acc_sh.at[sidx_v.at[s0]], add=True)
            pltpu.async_copy(z_hbm.at[gidx_v.at[s0 + 2]], rows0, sem0)
            wait1()
            pltpu.sync_copy(rows1, acc_sh.at[sidx_v.at[s0 + 1]], add=True)

        pltpu.async_copy(z_hbm.at[gidx_v.at[CPR2 - 1]], rows1, sem1)
        wait0()
        pltpu.sync_copy(rows0, acc_sh.at[sidx_v.at[CPR2 - 2]], add=True)
        wait1()
        pltpu.sync_copy(rows1, acc_sh.at[sidx_v.at[CPR2 - 1]], add=True)

    plsc.subcore_barrier()
    pltpu.sync_copy(acc_sh.at[pl.ds(sid * ZROWS, ZROWS)],
                    part_hbm.at[cid, pl.ds(sid * ZROWS, ZROWS)])


BM = 1000


def _mm_body(x_ref, w_ref, deg_ref, z_ref):
    total = deg_ref[:, 0:1] + deg_ref[:, 1:2] + 1.0
    dis = lax.rsqrt(total)
    h = lax.dot_general(x_ref[...], w_ref[...], (((1,), (1,)), ((), ())),
                        preferred_element_type=jnp.float32)
    z_ref[...] = h * dis


_mm_call = pl.pallas_call(
    _mm_body,
    grid=(N // BM,),
    in_specs=[
        pl.BlockSpec((BM, D), lambda i: (i, 0)),
        pl.BlockSpec((D, D), lambda i: (0, 0)),
        pl.BlockSpec((BM, NC), lambda i: (i, 0)),
    ],
    out_specs=pl.BlockSpec((BM, D), lambda i: (i, 0)),
    out_shape=jax.ShapeDtypeStruct((N, D), jnp.float32),
)


def _fin_body(p0_ref, p1_ref, z_ref, deg_ref, b_ref, y_ref):
    total = deg_ref[:, 0:1] + deg_ref[:, 1:2] + 1.0
    dis = lax.rsqrt(total)
    y_ref[...] = dis * (p0_ref[0] + p1_ref[0] + z_ref[...]) + b_ref[...]


_fin_call = pl.pallas_call(
    _fin_body,
    grid=(N // BM,),
    in_specs=[
        pl.BlockSpec((1, BM, D), lambda i: (0, i, 0)),
        pl.BlockSpec((1, BM, D), lambda i: (1, i, 0)),
        pl.BlockSpec((BM, D), lambda i: (i, 0)),
        pl.BlockSpec((BM, NC), lambda i: (i, 0)),
        pl.BlockSpec((1, D), lambda i: (0, 0)),
    ],
    out_specs=pl.BlockSpec((BM, D), lambda i: (i, 0)),
    out_shape=jax.ShapeDtypeStruct((N, D), jnp.float32),
)


def kernel(x, edge_index, W, b):
    src = edge_index[0]
    dst = edge_index[1]
    npad_e = EP - 2 * E
    gpad = (jnp.arange(npad_e, dtype=jnp.int32) * 64) % N
    spad = N + jnp.arange(npad_e, dtype=jnp.int32) % NDUMMY
    gidx4d = jnp.concatenate([dst, src, gpad]).reshape(NW, NCH2, CPR2, G2)
    sidx4d = jnp.concatenate([src, dst, spad]).reshape(NW, NCH2, CPR2, G2)

    deg_pad = _deg_call(sidx4d)
    deg_pair = jnp.transpose(deg_pad[:, 0, :N])
    z = _mm_call(x, W, deg_pair)
    part = _agg_call(z, gidx4d, sidx4d)
    y = _fin_call(part, part, z, deg_pair, b.reshape(1, D))
    return y


# speedup vs baseline: 1.0930x; 1.0517x over previous
import functools

import jax
import jax.numpy as jnp
from jax import lax
from jax.experimental import pallas as pl
from jax.experimental.pallas import tpu as pltpu
from jax.experimental.pallas import tpu_sc as plsc

N = 10000
E = 320000
D = 128
NC = 2
NS = 16
NW = NC * NS
G = 80
RG = 2 * E // G
RPT = RG // NW
NCH = 10
CPR = RPT // NCH
G2 = 128
NCH2 = 4
CPR2 = 40
G3 = 64
NCH3 = 8
CPR3 = 40
NBUF = 4
EP = NW * NCH2 * CPR2 * G2
NDUMMY = 240
NPAD = 10240
DPT = NPAD // NS
ZROWS = NPAD // NS
ZB = 32

_mesh = plsc.VectorSubcoreMesh(
    core_axis_name="c", subcore_axis_name="s", num_cores=NC, num_subcores=NS)


@functools.partial(
    pl.kernel,
    mesh=_mesh,
    out_type=jax.ShapeDtypeStruct((NC, 1, NPAD), jnp.float32),
    scratch_types=[
        pltpu.VMEM((CPR2, G2), jnp.int32),
        pltpu.VMEM((G2,), jnp.float32),
        pltpu.VMEM((DPT,), jnp.float32),
        pltpu.VMEM_SHARED((NPAD,), jnp.float32),
        pltpu.SemaphoreType.DMA,
    ],
)
def _deg_call(sidx_hbm, deg_hbm, idx_v, ones_v, zero_v, deg_sh, sem):
    cid = lax.axis_index("c")
    sid = lax.axis_index("s")
    wid = cid * NS + sid

    for k in range(DPT // 16):
        zero_v[pl.ds(k * 16, 16)] = jnp.zeros((16,), jnp.float32)
    for k in range(G2 // 16):
        ones_v[pl.ds(k * 16, 16)] = jnp.ones((16,), jnp.float32)
    pltpu.sync_copy(zero_v, deg_sh.at[pl.ds(sid * DPT, DPT)])
    plsc.subcore_barrier()

    @pl.loop(0, NCH2)
    def _(c):
        pltpu.sync_copy(sidx_hbm.at[wid, c], idx_v)

        @pl.loop(0, CPR2)
        def _(g):
            pltpu.async_copy(ones_v, deg_sh.at[idx_v.at[g]], sem, add=True)

        @pl.loop(0, CPR2)
        def _(g):
            pltpu.make_async_copy(ones_v, deg_sh.at[pl.ds(0, G2)], sem).wait()

    plsc.subcore_barrier()
    pltpu.sync_copy(deg_sh.at[pl.ds(sid * DPT, DPT)],
                    deg_hbm.at[cid, 0, pl.ds(sid * DPT, DPT)])


@functools.partial(
    pl.kernel,
    mesh=_mesh,
    out_type=jax.ShapeDtypeStruct((NC, NPAD, D), jnp.float32),
    scratch_types=[
        pltpu.VMEM((CPR3, G3), jnp.int32),
        pltpu.VMEM((CPR3, G3), jnp.int32),
        [pltpu.VMEM((G3, D), jnp.float32)] * NBUF,
        pltpu.VMEM((ZB, D), jnp.float32),
        pltpu.VMEM_SHARED((NPAD, D), jnp.float32),
        [pltpu.SemaphoreType.DMA] * NBUF,
    ],
)
def _agg_call(z_hbm, gidx_hbm, sidx_hbm, part_hbm,
              gidx_v, sidx_v, rows, zbuf_v, acc_sh, sems):
    cid = lax.axis_index("c")
    sid = lax.axis_index("s")
    wid = cid * NS + sid

    @pl.loop(0, ZB)
    def _(i):
        for k in range(D // 16):
            zbuf_v[i, pl.ds(k * 16, 16)] = jnp.zeros((16,), jnp.float32)

    for k in range(ZROWS // ZB):
        pltpu.sync_copy(zbuf_v, acc_sh.at[pl.ds(sid * ZROWS + k * ZB, ZB)])
    plsc.subcore_barrier()

    def wait(b):
        pltpu.make_async_copy(z_hbm.at[pl.ds(0, G3)], rows[b], sems[b]).wait()

    @pl.loop(0, NCH3)
    def _(c):
        pltpu.sync_copy(gidx_hbm.at[wid, c], gidx_v)
        pltpu.sync_copy(sidx_hbm.at[wid, c], sidx_v)
        for b in range(NBUF):
            pltpu.async_copy(z_hbm.at[gidx_v.at[b]], rows[b], sems[b])

        @pl.loop(0, CPR3 // NBUF - 1)
        def _(t):
            s0 = NBUF * t
            for b in range(NBUF):
                wait(b)
                pltpu.sync_copy(rows[b], acc_sh.at[sidx_v.at[s0 + b]], add=True)
                pltpu.async_copy(z_hbm.at[gidx_v.at[s0 + b + NBUF]], rows[b],
                                 sems[b])

        for b in range(NBUF):
            wait(b)
            pltpu.sync_copy(rows[b], acc_sh.at[sidx_v.at[CPR3 - NBUF + b]],
                            add=True)

    plsc.subcore_barrier()
    pltpu.sync_copy(acc_sh.at[pl.ds(sid * ZROWS, ZROWS)],
                    part_hbm.at[cid, pl.ds(sid * ZROWS, ZROWS)])


BM = 1000


def _mm_body(x_ref, w_ref, deg_ref, z_ref):
    total = deg_ref[:, 0:1] + deg_ref[:, 1:2] + 1.0
    dis = lax.rsqrt(total)
    h = lax.dot_general(x_ref[...], w_ref[...], (((1,), (1,)), ((), ())),
                        preferred_element_type=jnp.float32)
    z_ref[...] = h * dis


_mm_call = pl.pallas_call(
    _mm_body,
    grid=(N // BM,),
    in_specs=[
        pl.BlockSpec((BM, D), lambda i: (i, 0)),
        pl.BlockSpec((D, D), lambda i: (0, 0)),
        pl.BlockSpec((BM, NC), lambda i: (i, 0)),
    ],
    out_specs=pl.BlockSpec((BM, D), lambda i: (i, 0)),
    out_shape=jax.ShapeDtypeStruct((N, D), jnp.float32),
)


def _fin_body(p0_ref, p1_ref, z_ref, deg_ref, b_ref, y_ref):
    total = deg_ref[:, 0:1] + deg_ref[:, 1:2] + 1.0
    dis = lax.rsqrt(total)
    y_ref[...] = dis * (p0_ref[0] + p1_ref[0] + z_ref[...]) + b_ref[...]


_fin_call = pl.pallas_call(
    _fin_body,
    grid=(N // BM,),
    in_specs=[
        pl.BlockSpec((1, BM, D), lambda i: (0, i, 0)),
        pl.BlockSpec((1, BM, D), lambda i: (1, i, 0)),
        pl.BlockSpec((BM, D), lambda i: (i, 0)),
        pl.BlockSpec((BM, NC), lambda i: (i, 0)),
        pl.BlockSpec((1, D), lambda i: (0, 0)),
    ],
    out_specs=pl.BlockSpec((BM, D), lambda i: (i, 0)),
    out_shape=jax.ShapeDtypeStruct((N, D), jnp.float32),
)


def kernel(x, edge_index, W, b):
    src = edge_index[0]
    dst = edge_index[1]
    npad_e = EP - 2 * E
    gpad = (jnp.arange(npad_e, dtype=jnp.int32) * 64) % N
    spad = N + jnp.arange(npad_e, dtype=jnp.int32) % NDUMMY
    gidx4d = jnp.concatenate([dst, src, gpad]).reshape(NW, NCH2, CPR2, G2)
    sidx4d = jnp.concatenate([src, dst, spad]).reshape(NW, NCH2, CPR2, G2)
    gidx_agg = gidx4d.reshape(NW, NCH3, CPR3, G3)
    sidx_agg = sidx4d.reshape(NW, NCH3, CPR3, G3)

    deg_pad = _deg_call(sidx4d)
    deg_pair = jnp.transpose(deg_pad[:, 0, :N])
    z = _mm_call(x, W, deg_pair)
    part = _agg_call(z, gidx_agg, sidx_agg)
    y = _fin_call(part, part, z, deg_pair, b.reshape(1, D))
    return y


# speedup vs baseline: 1.0946x; 1.0015x over previous
import functools

import jax
import jax.numpy as jnp
from jax import lax
from jax.experimental import pallas as pl
from jax.experimental.pallas import tpu as pltpu
from jax.experimental.pallas import tpu_sc as plsc

N = 10000
E = 320000
D = 128
NC = 2
NS = 16
NW = NC * NS
G2 = 128
NCH2 = 4
CPR2 = 40
G3 = 64
NCH3 = 8
CPR3 = 40
NBUF = 4
EP = NW * NCH2 * CPR2 * G2
NDUMMY = 240
NPAD = 10240
DPT = NPAD // NS
ZROWS = NPAD // NS
ZB = 32

_mesh = plsc.VectorSubcoreMesh(
    core_axis_name="c", subcore_axis_name="s", num_cores=NC, num_subcores=NS)


@functools.partial(
    pl.kernel,
    mesh=_mesh,
    out_type=jax.ShapeDtypeStruct((NC, 1, NPAD), jnp.float32),
    scratch_types=[
        pltpu.VMEM((CPR2, G2), jnp.int32),
        pltpu.VMEM((G2,), jnp.float32),
        pltpu.VMEM((DPT,), jnp.float32),
        pltpu.VMEM_SHARED((NPAD,), jnp.float32),
        pltpu.SemaphoreType.DMA,
    ],
)
def _deg_call(sidx_hbm, deg_hbm, idx_v, ones_v, zero_v, deg_sh, sem):
    cid = lax.axis_index("c")
    sid = lax.axis_index("s")
    wid = cid * NS + sid

    for k in range(DPT // 16):
        zero_v[pl.ds(k * 16, 16)] = jnp.zeros((16,), jnp.float32)
    for k in range(G2 // 16):
        ones_v[pl.ds(k * 16, 16)] = jnp.ones((16,), jnp.float32)
    pltpu.sync_copy(zero_v, deg_sh.at[pl.ds(sid * DPT, DPT)])
    plsc.subcore_barrier()

    @pl.loop(0, NCH2)
    def _(c):
        pltpu.sync_copy(sidx_hbm.at[wid, c], idx_v)

        @pl.loop(0, CPR2)
        def _(g):
            pltpu.async_copy(ones_v, deg_sh.at[idx_v.at[g]], sem, add=True)

        @pl.loop(0, CPR2)
        def _(g):
            pltpu.make_async_copy(ones_v, deg_sh.at[pl.ds(0, G2)], sem).wait()

    plsc.subcore_barrier()
    pltpu.sync_copy(deg_sh.at[pl.ds(sid * DPT, DPT)],
                    deg_hbm.at[cid, 0, pl.ds(sid * DPT, DPT)])


@functools.partial(
    pl.kernel,
    mesh=_mesh,
    out_type=jax.ShapeDtypeStruct((NC, NPAD, D), jnp.float32),
    scratch_types=[
        pltpu.VMEM((CPR3, G3), jnp.int32),
        pltpu.VMEM((CPR3, G3), jnp.int32),
        [pltpu.VMEM((G3, D), jnp.float32)] * NBUF,
        pltpu.VMEM((ZB, D), jnp.float32),
        pltpu.VMEM_SHARED((NPAD, D), jnp.float32),
        [pltpu.SemaphoreType.DMA] * NBUF,
    ],
)
def _agg_call(z_hbm, gidx_hbm, sidx_hbm, part_hbm,
              gidx_v, sidx_v, rows, zbuf_v, acc_sh, sems):
    cid = lax.axis_index("c")
    sid = lax.axis_index("s")
    wid = cid * NS + sid

    @pl.loop(0, ZB)
    def _(i):
        for k in range(D // 16):
            zbuf_v[i, pl.ds(k * 16, 16)] = jnp.zeros((16,), jnp.float32)

    for k in range(ZROWS // ZB):
        pltpu.sync_copy(zbuf_v, acc_sh.at[pl.ds(sid * ZROWS + k * ZB, ZB)])
    plsc.subcore_barrier()

    def wait(b):
        pltpu.make_async_copy(z_hbm.at[pl.ds(0, G3)], rows[b], sems[b]).wait()

    @pl.loop(0, NCH3)
    def _(c):
        pltpu.sync_copy(gidx_hbm.at[wid, c], gidx_v)
        pltpu.sync_copy(sidx_hbm.at[wid, c], sidx_v)
        for b in range(NBUF):
            pltpu.async_copy(z_hbm.at[gidx_v.at[b]], rows[b], sems[b])

        @pl.loop(0, CPR3 // NBUF - 1)
        def _(t):
            s0 = NBUF * t
            for b in range(NBUF):
                wait(b)
                pltpu.sync_copy(rows[b], acc_sh.at[sidx_v.at[s0 + b]], add=True)
                pltpu.async_copy(z_hbm.at[gidx_v.at[s0 + b + NBUF]], rows[b],
                                 sems[b])

        for b in range(NBUF):
            wait(b)
            pltpu.sync_copy(rows[b], acc_sh.at[sidx_v.at[CPR3 - NBUF + b]],
                            add=True)

    plsc.subcore_barrier()
    pltpu.sync_copy(acc_sh.at[pl.ds(sid * ZROWS, ZROWS)],
                    part_hbm.at[cid, pl.ds(sid * ZROWS, ZROWS)])


BM = 1000


def _mm_body(x_ref, w_ref, deg_ref, z_ref):
    total = deg_ref[:, 0:1] + deg_ref[:, 1:2] + 1.0
    dis = lax.rsqrt(total)
    h = lax.dot_general(x_ref[...], w_ref[...], (((1,), (1,)), ((), ())),
                        preferred_element_type=jnp.float32)
    z_ref[...] = h * dis


_mm_call = pl.pallas_call(
    _mm_body,
    grid=(N // BM,),
    in_specs=[
        pl.BlockSpec((BM, D), lambda i: (i, 0)),
        pl.BlockSpec((D, D), lambda i: (0, 0)),
        pl.BlockSpec((BM, NC), lambda i: (i, 0)),
    ],
    out_specs=pl.BlockSpec((BM, D), lambda i: (i, 0)),
    out_shape=jax.ShapeDtypeStruct((N, D), jnp.float32),
)


def _fin_body(p0_ref, p1_ref, z_ref, deg_ref, b_ref, y_ref):
    total = deg_ref[:, 0:1] + deg_ref[:, 1:2] + 1.0
    dis = lax.rsqrt(total)
    y_ref[...] = dis * (p0_ref[0] + p1_ref[0] + z_ref[...]) + b_ref[...]


_fin_call = pl.pallas_call(
    _fin_body,
    grid=(N // BM,),
    in_specs=[
        pl.BlockSpec((1, BM, D), lambda i: (0, i, 0)),
        pl.BlockSpec((1, BM, D), lambda i: (1, i, 0)),
        pl.BlockSpec((BM, D), lambda i: (i, 0)),
        pl.BlockSpec((BM, NC), lambda i: (i, 0)),
        pl.BlockSpec((1, D), lambda i: (0, 0)),
    ],
    out_specs=pl.BlockSpec((BM, D), lambda i: (i, 0)),
    out_shape=jax.ShapeDtypeStruct((N, D), jnp.float32),
)


def kernel(x, edge_index, W, b):
    src = edge_index[0]
    dst = edge_index[1]
    npad_e = EP - 2 * E
    gpad = (jnp.arange(npad_e, dtype=jnp.int32) * 64) % N
    spad = N + jnp.arange(npad_e, dtype=jnp.int32) % NDUMMY
    gidx4d = jnp.concatenate([dst, src, gpad]).reshape(NW, NCH2, CPR2, G2)
    sidx4d = jnp.concatenate([src, dst, spad]).reshape(NW, NCH2, CPR2, G2)
    gidx_agg = gidx4d.reshape(NW, NCH3, CPR3, G3)
    sidx_agg = sidx4d.reshape(NW, NCH3, CPR3, G3)

    deg_pad = _deg_call(sidx4d)
    deg_pair = jnp.transpose(deg_pad[:, 0, :N])
    z = _mm_call(x, W, deg_pair)
    part = _agg_call(z, gidx_agg, sidx_agg)
    y = _fin_call(part, part, z, deg_pair, b.reshape(1, D))
    return y
